# baseline (speedup 1.0000x reference)
import functools

import jax
import jax.numpy as jnp
from jax import lax
from jax.experimental import pallas as pl
from jax.experimental.pallas import tpu as pltpu
from jax.experimental.pallas import tpu_sc as plsc

N = 10000
E = 320000
D = 128
H1 = 256
EMB = 128
MLP_H = 64
B = 16384
LAM = 0.5

NC = 2
NS = 16
FCH = 128
EPT = E // NS
G = 80
NIT = EPT // G
BSTRIDE = 256
NITP = 252
BW3 = 3 * G
NP = 10240
RPT = NP // NS

_mesh = plsc.VectorSubcoreMesh(core_axis_name="c", subcore_axis_name="s")


def _make_spmm(C):
    CPS = C // NC

    @functools.partial(
        pl.kernel,
        out_type=jax.ShapeDtypeStruct((C * NP, FCH), jnp.float32),
        mesh=_mesh,
        scratch_types=(
            [pltpu.VMEM((BW3,), jnp.float32)] * 4
            + [pltpu.VMEM((G, FCH), jnp.float32)] * 4
            + [pltpu.VMEM((G,), jnp.int32)] * 4
            + [pltpu.VMEM((G,), jnp.int32)] * 2
            + [pltpu.VMEM_SHARED((NP, FCH), jnp.float32)]
            + [pltpu.SemaphoreType.DMA] * 10
        ),
    )
    def spmm(x_hbm, ei_hbm, z_hbm, out_hbm,
             i0, i1, i2, i3, r0, r1, r2, r3, d0, d1, d2, d3, s0, s1, acc,
             si0, si1, si2, si3, sg0, sg1, ss0, ss1, ss2, ss3):
        idx = [i0, i1, i2, i3]
        rows = [r0, r1, r2, r3]
        dsti = [d0, d1, d2, d3]
        srci = [s0, s1]
        si = [si0, si1, si2, si3]
        sg = [sg0, sg1]
        ss = [ss0, ss1, ss2, ss3]

        cid = lax.axis_index("c")
        sid = lax.axis_index("s")
        b0 = sid * BSTRIDE
        rbase = sid * RPT

        def ei_src(blk):
            return ei_hbm.at[pl.ds(blk * BW3, BW3)]

        def prep(gq, q2, q4, off, guard_cond):
            pltpu.make_async_copy(ei_src(b0 + gq), idx[q4], si[q4]).wait()

            @plsc.parallel_loop(0, G // 16)
            def addoff(i):
                srci[q2][pl.ds(i * 16, 16)] = (
                    idx[q4][pl.ds(G + i * 16, 16)].astype(jnp.int32) + off)

            if guard_cond is not False:
                def _w():
                    pltpu.make_async_copy(
                        rows[q4], acc.at[dsti[q4]], ss[q4]).wait()
                if guard_cond is None:
                    _w()
                else:
                    pl.when(guard_cond)(_w)
            pltpu.async_copy(x_hbm.at[srci[q2]], rows[q4], sg[q2])

        def cstep(p):
            p2 = p % 2
            pltpu.make_async_copy(x_hbm.at[srci[p2]], rows[p], sg[p2]).wait()

            @plsc.parallel_loop(0, G // 16)
            def edge16(gi):
                dsti[p][pl.ds(gi * 16, 16)] = (
                    idx[p][pl.ds(gi * 16, 16)].astype(jnp.int32))
                vv = idx[p][pl.ds(2 * G + gi * 16, 16)]
                for l in range(16):
                    v = vv[l]
                    e = gi * 16 + l
                    for k in range(FCH // 16):
                        sl = pl.ds(k * 16, 16)
                        rows[p][e, sl] = rows[p][e, sl] * v

            pltpu.async_copy(rows[p], acc.at[dsti[p]], ss[p], add=True)

        for j in range(CPS):
            c = cid * CPS + j
            goff = c * N
            row0 = c * NP
            off = jnp.full((16,), goff, jnp.int32)
            pltpu.sync_copy(z_hbm, acc.at[pl.ds(rbase, RPT)])
            plsc.subcore_barrier()

            for p in range(4):
                pltpu.async_copy(ei_src(b0 + p), idx[p], si[p])
            prep(0, 0, 0, off, False)
            prep(1, 1, 1, off, False)

            def body(o4, carry):
                gb = 4 * o4
                for p in range(4):
                    g = gb + p
                    cstep(p)
                    guard = (o4 > 0) if p < 2 else None
                    prep(g + 2, p % 2, (p + 2) % 4, off, guard)
                    pltpu.async_copy(ei_src(b0 + g + 4), idx[p], si[p])
                return carry

            lax.fori_loop(0, NITP // 4, body, 0)
            pltpu.make_async_copy(x_hbm.at[srci[0]], rows[0], sg[0]).wait()
            pltpu.make_async_copy(x_hbm.at[srci[1]], rows[1], sg[1]).wait()
            pltpu.make_async_copy(rows[2], acc.at[dsti[2]], ss[2]).wait()
            pltpu.make_async_copy(rows[3], acc.at[dsti[3]], ss[3]).wait()
            pltpu.make_async_copy(ei_src(b0 + NITP + 2), idx[2], si[2]).wait()
            pltpu.make_async_copy(ei_src(b0 + NITP + 3), idx[3], si[3]).wait()

            plsc.subcore_barrier()
            pltpu.sync_copy(acc.at[pl.ds(rbase, RPT)],
                            out_hbm.at[pl.ds(row0 + rbase, RPT)])
            if j + 1 < CPS:
                plsc.subcore_barrier()

    return spmm


_spmm4 = _make_spmm(4)
_spmm2 = _make_spmm(2)

PG = 2 * B // (NC * NS)
PGC = 128


@functools.partial(
    pl.kernel,
    out_type=jax.ShapeDtypeStruct((2 * B, EMB), jnp.float32),
    mesh=_mesh,
    scratch_types=[
        pltpu.VMEM((PGC,), jnp.int32),
        pltpu.VMEM((PGC, EMB), jnp.float32),
        pltpu.SemaphoreType.DMA,
    ],
)
def _pair_gather(h_hbm, idx_hbm, out_hbm, idx_v, rows_v, sem):
    wid = lax.axis_index("s") * NC + lax.axis_index("c")
    base = wid * PG

    def it(g, carry):
        b0 = base + g * PGC
        pltpu.sync_copy(idx_hbm.at[pl.ds(b0, PGC)], idx_v)
        pltpu.async_copy(h_hbm.at[idx_v], rows_v, sem).wait()
        pltpu.sync_copy(rows_v, out_hbm.at[pl.ds(b0, PGC)])
        return carry

    lax.fori_loop(0, PG // PGC, it, 0)


_NT = 5
_RB = N // _NT


def _mm_in_body(x_ref, w_ref, o_ref):
    o_ref[...] = jnp.dot(x_ref[0], w_ref[0],
                         preferred_element_type=jnp.float32)[None, None]


def _mm_in(x, w):
    return pl.pallas_call(
        _mm_in_body,
        grid=(2, _NT, 2),
        in_specs=[
            pl.BlockSpec((1, _RB, D), lambda b, i, h: (b, i, 0)),
            pl.BlockSpec((1, D, FCH), lambda b, i, h: (b, 0, h)),
        ],
        out_specs=pl.BlockSpec((1, 1, _RB, FCH), lambda b, i, h: (b, h, i, 0)),
        out_shape=jax.ShapeDtypeStruct((2, 2, N, FCH), jnp.float32),
    )(x, w)


def _mm_mid_body(h_ref, w_ref, o_ref):
    a = jnp.maximum(h_ref[...], 0.0)
    w = w_ref[0]
    o_ref[...] = (jnp.dot(a[0], w[:FCH], preferred_element_type=jnp.float32)
                  + jnp.dot(a[1], w[FCH:], preferred_element_type=jnp.float32))[None]


def _mm_mid(h1, w):
    return pl.pallas_call(
        _mm_mid_body,
        grid=(2, _NT),
        in_specs=[
            pl.BlockSpec((2, _RB, FCH), lambda b, i: (b, i, 0)),
            pl.BlockSpec((1, H1, EMB), lambda b, i: (b, 0, 0)),
        ],
        out_specs=pl.BlockSpec((1, _RB, EMB), lambda b, i: (b, i, 0)),
        out_shape=jax.ShapeDtypeStruct((2, N, EMB), jnp.float32),
    )(h1, w)


def _combine_body(h_ref, o_ref):
    a = h_ref[...]
    o_ref[...] = ((1.0 - LAM) * jnp.maximum(a[0], 0.0)
                  + LAM * jnp.maximum(a[1], 0.0))


def _combine(h2):
    return pl.pallas_call(
        _combine_body,
        grid=(_NT,),
        in_specs=[pl.BlockSpec((2, _RB, EMB), lambda i: (0, i, 0))],
        out_specs=pl.BlockSpec((_RB, EMB), lambda i: (i, 0)),
        out_shape=jax.ShapeDtypeStruct((N, EMB), jnp.float32),
    )(h2)


_BB = 2048


def _mlp_body(xp_ref, w1_ref, b1_ref, w2_ref, o_ref):
    e = xp_ref[...]
    hh = (jnp.dot(e[0], w1_ref[:EMB], preferred_element_type=jnp.float32)
          + jnp.dot(e[1], w1_ref[EMB:], preferred_element_type=jnp.float32)
          + b1_ref[...])
    hh = jnp.maximum(hh, 0.0)
    p = jnp.dot(hh, w2_ref[...], preferred_element_type=jnp.float32)
    o_ref[...] = jnp.maximum(p, 0.0)


def _mlp(xp, w1, b1, w2):
    return pl.pallas_call(
        _mlp_body,
        grid=(B // _BB,),
        in_specs=[
            pl.BlockSpec((2, _BB, EMB), lambda i: (0, i, 0)),
            pl.BlockSpec((2 * EMB, MLP_H), lambda i: (0, 0)),
            pl.BlockSpec((1, MLP_H), lambda i: (0, 0)),
            pl.BlockSpec((MLP_H, 1), lambda i: (0, 0)),
        ],
        out_specs=pl.BlockSpec((_BB, 1), lambda i: (i, 0)),
        out_shape=jax.ShapeDtypeStruct((B, 1), jnp.float32),
    )(xp, w1, b1, w2)


def kernel(edge_index, adj_vals, train_sample, rna, atac,
           W_rna1, W_rna2, W_atac1, W_atac2, mlp_w1, mlp_b1, mlp_w2):
    ei3 = jnp.stack([edge_index[0].astype(jnp.float32),
                     edge_index[1].astype(jnp.float32), adj_vals])
    eiT = ei3.reshape(3, NS, NIT, G)
    eiT = jnp.pad(eiT, ((0, 0), (0, 0), (0, BSTRIDE - NIT), (0, 0)))
    eiB = eiT.transpose(1, 2, 0, 3).reshape(-1)
    x_in = jnp.stack([rna, atac])
    w1s = jnp.stack([W_rna1, W_atac1])
    w2s = jnp.stack([W_rna2, W_atac2])

    zrows = jnp.zeros((RPT, FCH), jnp.float32)

    X1 = _mm_in(x_in, w1s)
    H1f = _spmm4(X1.reshape(4 * N, FCH), eiB, zrows)
    X2 = _mm_mid(H1f.reshape(4, NP, FCH), w2s)
    H2f = _spmm2(X2.reshape(2 * N, FCH), eiB, zrows)
    h = _combine(H2f.reshape(2, NP, EMB))
    idxp = train_sample.T.reshape(-1)
    xp = _pair_gather(h, idxp)
    return _mlp(xp.reshape(2, B, EMB), mlp_w1, mlp_b1.reshape(1, MLP_H), mlp_w2)


# speedup vs baseline: 2.1721x; 2.1721x over previous
import functools

import jax
import jax.numpy as jnp
from jax import lax
from jax.experimental import pallas as pl
from jax.experimental.pallas import tpu as pltpu
from jax.experimental.pallas import tpu_sc as plsc

N = 10000
E = 320000
D = 128
H1 = 256
EMB = 128
MLP_H = 64
B = 16384
LAM = 0.5

NC = 2
NS = 16
FCH = 128
EPT = E // NS
G = 80
NIT = EPT // G
NBLK = E // G
BW3 = 3 * G
NP = 10240
RPT = NP // NS

_mesh = plsc.VectorSubcoreMesh(core_axis_name="c", subcore_axis_name="s")


@functools.partial(
    pl.kernel,
    out_type=jax.ShapeDtypeStruct((2 * NP, FCH), jnp.float32),
    mesh=_mesh,
    scratch_types=[
        pltpu.VMEM((BW3,), jnp.float32),
        pltpu.VMEM((BW3,), jnp.float32),
        pltpu.VMEM((G,), jnp.int32),
        pltpu.VMEM((G,), jnp.int32),
        pltpu.VMEM((G,), jnp.int32),
        pltpu.VMEM((G,), jnp.int32),
        pltpu.VMEM((G, FCH), jnp.float32),
        pltpu.VMEM((G, FCH), jnp.float32),
        pltpu.VMEM((G, FCH), jnp.float32),
        pltpu.VMEM((G, FCH), jnp.float32),
        pltpu.VMEM_SHARED((NP, FCH), jnp.float32),
        pltpu.SemaphoreType.DMA,
        pltpu.SemaphoreType.DMA,
        pltpu.SemaphoreType.DMA,
        pltpu.SemaphoreType.DMA,
        pltpu.SemaphoreType.DMA,
        pltpu.SemaphoreType.DMA,
    ],
)
def _spmm2(x_hbm, ei_hbm, z_hbm, out_hbm,
           idx0, idx1, srci0, srci1, dsti0, dsti1,
           rows0, rows1, sbuf0, sbuf1, acc,
           si0, si1, sg0, sg1, ss0, ss1):
    cid = lax.axis_index("c")
    sid = lax.axis_index("s")

    b0 = sid * NIT
    rbase = sid * RPT

    def stage_i(idx_ref, sem, blk):
        pltpu.async_copy(ei_hbm.at[pl.ds(blk * BW3, BW3)], idx_ref, sem)

    def stage_p(idx_ref, sem, srci_ref, rows_ref, gsem, blk, off):
        pltpu.make_async_copy(
            ei_hbm.at[pl.ds(blk * BW3, BW3)], idx_ref, sem).wait()

        @plsc.parallel_loop(0, G // 16)
        def addoff(i):
            sl16 = pl.ds(G + i * 16, 16)
            srci_ref[pl.ds(i * 16, 16)] = (
                idx_ref[sl16].astype(jnp.int32) + off)
        pltpu.async_copy(x_hbm.at[srci_ref], rows_ref, gsem)

    def scat_wait(sbuf_ref, dsti_ref, ssem):
        pltpu.make_async_copy(sbuf_ref, acc.at[dsti_ref], ssem).wait()

    def stage_c(idx_ref, srci_ref, dsti_ref, rows_ref, sbuf_ref,
                gsem, ssem, guard_cond):
        pltpu.make_async_copy(x_hbm.at[srci_ref], rows_ref, gsem).wait()
        if guard_cond is None:
            scat_wait(sbuf_ref, dsti_ref, ssem)
        else:
            @pl.when(guard_cond)
            def _():
                scat_wait(sbuf_ref, dsti_ref, ssem)

        @plsc.parallel_loop(0, G // 16)
        def edge16(gi):
            dsti_ref[pl.ds(gi * 16, 16)] = (
                idx_ref[pl.ds(gi * 16, 16)].astype(jnp.int32))
            vv = idx_ref[pl.ds(2 * G + gi * 16, 16)]
            for l in range(16):
                v = vv[l]
                e = gi * 16 + l
                for k in range(FCH // 16):
                    sl = pl.ds(k * 16, 16)
                    sbuf_ref[e, sl] = rows_ref[e, sl] * v
        pltpu.async_copy(sbuf_ref, acc.at[dsti_ref], ssem, add=True)

    goff = cid * N
    row0 = cid * NP
    off = jnp.full((16,), goff, jnp.int32)
    pltpu.sync_copy(z_hbm, acc.at[pl.ds(rbase, RPT)])
    plsc.subcore_barrier()

    stage_i(idx0, si0, b0)
    stage_i(idx1, si1, b0 + 1)
    stage_p(idx0, si0, srci0, rows0, sg0, b0, off)

    def body(o2, carry):
        g0 = 2 * o2
        guard = o2 > 0
        stage_p(idx1, si1, srci1, rows1, sg1, b0 + g0 + 1, off)
        stage_c(idx0, srci0, dsti0, rows0, sbuf0, sg0, ss0, guard)
        stage_i(idx0, si0, b0 + g0 + 2)
        stage_p(idx0, si0, srci0, rows0, sg0, b0 + g0 + 2, off)
        stage_c(idx1, srci1, dsti1, rows1, sbuf1, sg1, ss1, guard)
        stage_i(idx1, si1, b0 + g0 + 3)
        return carry

    lax.fori_loop(0, NIT // 2 - 1, body, 0)
    stage_p(idx1, si1, srci1, rows1, sg1, b0 + NIT - 1, off)
    stage_c(idx0, srci0, dsti0, rows0, sbuf0, sg0, ss0, None)
    stage_c(idx1, srci1, dsti1, rows1, sbuf1, sg1, ss1, None)
    scat_wait(sbuf0, dsti0, ss0)
    scat_wait(sbuf1, dsti1, ss1)

    plsc.subcore_barrier()
    pltpu.sync_copy(acc.at[pl.ds(rbase, RPT)],
                    out_hbm.at[pl.ds(row0 + rbase, RPT)])


PG = 2 * B // (NC * NS)
PGC = 128


@functools.partial(
    pl.kernel,
    out_type=jax.ShapeDtypeStruct((2 * B, EMB), jnp.float32),
    mesh=_mesh,
    scratch_types=[
        pltpu.VMEM((PGC,), jnp.int32),
        pltpu.VMEM((PGC, EMB), jnp.float32),
        pltpu.SemaphoreType.DMA,
    ],
)
def _pair_gather(h_hbm, idx_hbm, out_hbm, idx_v, rows_v, sem):
    wid = lax.axis_index("s") * NC + lax.axis_index("c")
    base = wid * PG

    def it(g, carry):
        b0 = base + g * PGC
        pltpu.sync_copy(idx_hbm.at[pl.ds(b0, PGC)], idx_v)
        pltpu.async_copy(h_hbm.at[idx_v], rows_v, sem).wait()
        pltpu.sync_copy(rows_v, out_hbm.at[pl.ds(b0, PGC)])
        return carry

    lax.fori_loop(0, PG // PGC, it, 0)


_NT = 5
_RB = N // _NT


def _mm_fused_body(s_ref, w1_ref, w2_ref, o_ref):
    a = jnp.maximum(jnp.dot(s_ref[0], w1_ref[0],
                            preferred_element_type=jnp.float32), 0.0)
    o_ref[...] = jnp.dot(a, w2_ref[0],
                         preferred_element_type=jnp.float32)[None]


def _mm_fused(s1, w1, w2):
    return pl.pallas_call(
        _mm_fused_body,
        grid=(2, _NT),
        in_specs=[
            pl.BlockSpec((1, _RB, FCH), lambda b, i: (b, i, 0)),
            pl.BlockSpec((1, D, H1), lambda b, i: (b, 0, 0)),
            pl.BlockSpec((1, H1, EMB), lambda b, i: (b, 0, 0)),
        ],
        out_specs=pl.BlockSpec((1, _RB, EMB), lambda b, i: (b, i, 0)),
        out_shape=jax.ShapeDtypeStruct((2, N, EMB), jnp.float32),
    )(s1, w1, w2)


def _combine_body(h_ref, o_ref):
    a = h_ref[...]
    o_ref[...] = ((1.0 - LAM) * jnp.maximum(a[0], 0.0)
                  + LAM * jnp.maximum(a[1], 0.0))


def _combine(h2):
    return pl.pallas_call(
        _combine_body,
        grid=(_NT,),
        in_specs=[pl.BlockSpec((2, _RB, EMB), lambda i: (0, i, 0))],
        out_specs=pl.BlockSpec((_RB, EMB), lambda i: (i, 0)),
        out_shape=jax.ShapeDtypeStruct((N, EMB), jnp.float32),
    )(h2)


_BB = 2048


def _mlp_body(xp_ref, w1_ref, b1_ref, w2_ref, o_ref):
    e = xp_ref[...]
    hh = (jnp.dot(e[0], w1_ref[:EMB], preferred_element_type=jnp.float32)
          + jnp.dot(e[1], w1_ref[EMB:], preferred_element_type=jnp.float32)
          + b1_ref[...])
    hh = jnp.maximum(hh, 0.0)
    p = jnp.dot(hh, w2_ref[...], preferred_element_type=jnp.float32)
    o_ref[...] = jnp.maximum(p, 0.0)


def _mlp(xp, w1, b1, w2):
    return pl.pallas_call(
        _mlp_body,
        grid=(B // _BB,),
        in_specs=[
            pl.BlockSpec((2, _BB, EMB), lambda i: (0, i, 0)),
            pl.BlockSpec((2 * EMB, MLP_H), lambda i: (0, 0)),
            pl.BlockSpec((1, MLP_H), lambda i: (0, 0)),
            pl.BlockSpec((MLP_H, 1), lambda i: (0, 0)),
        ],
        out_specs=pl.BlockSpec((_BB, 1), lambda i: (i, 0)),
        out_shape=jax.ShapeDtypeStruct((B, 1), jnp.float32),
    )(xp, w1, b1, w2)


def kernel(edge_index, adj_vals, train_sample, rna, atac,
           W_rna1, W_rna2, W_atac1, W_atac2, mlp_w1, mlp_b1, mlp_w2):
    ei3 = jnp.stack([edge_index[0].astype(jnp.float32),
                     edge_index[1].astype(jnp.float32), adj_vals])
    eiB = ei3.reshape(3, NBLK, G).transpose(1, 0, 2).reshape(-1)
    w1s = jnp.stack([W_rna1, W_atac1])
    w2s = jnp.stack([W_rna2, W_atac2])
    zrows = jnp.zeros((RPT, FCH), jnp.float32)

    x1f = jnp.concatenate([rna, atac], axis=0)
    S1 = _spmm2(x1f, eiB, zrows)
    X2 = _mm_fused(S1.reshape(2, NP, FCH), w1s, w2s)
    S2 = _spmm2(X2.reshape(2 * N, FCH), eiB, zrows)
    h = _combine(S2.reshape(2, NP, EMB))
    idxp = train_sample.T.reshape(-1)
    xp = _pair_gather(h, idxp)
    return _mlp(xp.reshape(2, B, EMB), mlp_w1, mlp_b1.reshape(1, MLP_H), mlp_w2)
